# Initial kernel scaffold; baseline (speedup 1.0000x reference)
import jax
import jax.numpy as jnp
from jax.experimental import pallas as pl


def kernel(x, edge_index, W1, W2, b2):
    raise NotImplementedError("write your pallas kernel here")


# speedup vs baseline: 14.4143x; 14.4143x over previous
import functools

import jax
import jax.numpy as jnp
from jax import lax
from jax.experimental import pallas as pl
from jax.experimental.pallas import tpu as pltpu
from jax.experimental.pallas import tpu_sc as plsc

N_NODES = 10000
N_EDGES = 320000
NFEAT = 128
NHID = 128
NCLASS = 40

NC = 2
NS = 16
NW = NC * NS
EDGES_PER_TILE = N_EDGES // NW
CHUNK = 80
NCHUNK = EDGES_PER_TILE // CHUNK
ROWS_PER_TILE = N_NODES // NS
ZR = 25

_mesh = functools.partial(
    plsc.VectorSubcoreMesh, core_axis_name="c", subcore_axis_name="s")


def _zero_fill(zb, d):
    z16 = jnp.zeros((16,), jnp.float32)
    for r in range(ZR):
        for q in range(d // 16):
            zb[r, pl.ds(16 * q, 16)] = z16


def _zero_acc(zb, acc, s):
    for j in range(ROWS_PER_TILE // ZR):
        pltpu.sync_copy(zb, acc.at[pl.ds(s * ROWS_PER_TILE + j * ZR, ZR)])


def _copy_out(acc, out, c, s):
    pltpu.sync_copy(acc.at[pl.ds(s * ROWS_PER_TILE, ROWS_PER_TILE)],
                    out.at[c, s])


def _deg_body(dst_hbm, out_hbm, dst_v, ones_v, zb, acc):
    c = lax.axis_index("c")
    s = lax.axis_index("s")
    wid = s * NC + c
    o16 = jnp.ones((16,), jnp.float32)
    for r in range(CHUNK):
        for q in range(NHID // 16):
            ones_v[r, pl.ds(16 * q, 16)] = o16
    _zero_fill(zb, NHID)
    _zero_acc(zb, acc, s)
    pltpu.sync_copy(dst_hbm.at[wid], dst_v)
    plsc.subcore_barrier()

    def body(i, carry):
        pltpu.sync_copy(ones_v, acc.at[dst_v.at[i]], add=True)
        return carry

    lax.fori_loop(0, NCHUNK, body, 0)
    plsc.subcore_barrier()
    _copy_out(acc, out_hbm, c, s)


def _deg(dst3):
    kern = pl.kernel(
        _deg_body,
        out_type=jax.ShapeDtypeStruct((NC, NS, ROWS_PER_TILE, NHID),
                                      jnp.float32),
        mesh=_mesh(),
        scratch_types=[
            pltpu.VMEM((NCHUNK, CHUNK), jnp.int32),
            pltpu.VMEM((CHUNK, NHID), jnp.float32),
            pltpu.VMEM((ZR, NHID), jnp.float32),
            pltpu.VMEM_SHARED((N_NODES, NHID), jnp.float32),
        ],
    )
    return kern(dst3).reshape(NC, N_NODES, NHID)


def _spmm_body(d, feat_hbm, src_hbm, dst_hbm, out_hbm,
               src_v, dst_v, rows_v, zb, acc, sem):
    c = lax.axis_index("c")
    s = lax.axis_index("s")
    wid = s * NC + c
    _zero_fill(zb, d)
    _zero_acc(zb, acc, s)
    pltpu.sync_copy(src_hbm.at[wid], src_v)
    pltpu.sync_copy(dst_hbm.at[wid], dst_v)
    plsc.subcore_barrier()

    def body(i, carry):
        pltpu.async_copy(feat_hbm.at[src_v.at[i]], rows_v, sem).wait()
        pltpu.sync_copy(rows_v, acc.at[dst_v.at[i]], add=True)
        return carry

    lax.fori_loop(0, NCHUNK, body, 0)
    plsc.subcore_barrier()
    _copy_out(acc, out_hbm, c, s)


def _spmm(feat, src3, dst3, d):
    kern = pl.kernel(
        functools.partial(_spmm_body, d),
        out_type=jax.ShapeDtypeStruct((NC, NS, ROWS_PER_TILE, d),
                                      jnp.float32),
        mesh=_mesh(),
        scratch_types=[
            pltpu.VMEM((NCHUNK, CHUNK), jnp.int32),
            pltpu.VMEM((NCHUNK, CHUNK), jnp.int32),
            pltpu.VMEM((CHUNK, d), jnp.float32),
            pltpu.VMEM((ZR, d), jnp.float32),
            pltpu.VMEM_SHARED((N_NODES, d), jnp.float32),
            pltpu.SemaphoreType.DMA,
        ],
    )
    return kern(feat, src3, dst3).reshape(NC, N_NODES, d)


ROWS_TC = 2000


def _scale_in_body(x_ref, w1_ref, degc_ref, xws_ref, dinv_ref):
    deg = jnp.maximum(degc_ref[0] + degc_ref[1], 1.0)
    dinv = lax.rsqrt(deg)
    dinv_ref[...] = dinv
    xw = jnp.dot(x_ref[...], w1_ref[...],
                 preferred_element_type=jnp.float32)
    xws_ref[...] = xw * dinv[:, 0:1]


def _mid_body(p_ref, dinv_ref, out_ref):
    dv = dinv_ref[:, 0:1]
    h = jnp.maximum((p_ref[0] + p_ref[1]) * dv, 0.0)
    out_ref[...] = h * dv


def _final_body(q_ref, dinv_ref, w2_ref, b2_ref, out_ref):
    z = (q_ref[0] + q_ref[1]) * dinv_ref[:, 0:1]
    logits = jnp.dot(z, w2_ref[...],
                     preferred_element_type=jnp.float32) + b2_ref[0:1, :]
    mx = jnp.max(logits, axis=1, keepdims=True)
    lse = jnp.log(jnp.sum(jnp.exp(logits - mx), axis=1, keepdims=True)) + mx
    out_ref[...] = logits - lse


_DBG_JNP_DEG = False
_DBG_JNP_SPMM = False


def kernel(x, edge_index, W1, W2, b2):
    ei = edge_index.astype(jnp.int32)
    src3 = ei[0].reshape(NW, NCHUNK, CHUNK)
    dst3 = ei[1].reshape(NW, NCHUNK, CHUNK)
    b2r = b2.reshape(1, NCLASS)

    if _DBG_JNP_DEG:
        degj = jax.ops.segment_sum(jnp.ones((N_EDGES,), jnp.float32),
                                   ei[1], num_segments=N_NODES)
        degc = jnp.stack([jnp.tile(degj[:, None], (1, 16)),
                          jnp.zeros((N_NODES, 16), jnp.float32)])
    else:
        degc = _deg(dst3)[:, :, :16]

    def _spmm_dbg(feat, a, b, dd):
        if _DBG_JNP_SPMM:
            acc = jax.ops.segment_sum(feat[ei[0]], ei[1],
                                      num_segments=N_NODES)
            return jnp.stack([acc, jnp.zeros_like(acc)])
        return _spmm(feat, a, b, dd)

    grid = (N_NODES // ROWS_TC,)
    xws, dinv16 = pl.pallas_call(
        _scale_in_body,
        grid=grid,
        in_specs=[
            pl.BlockSpec((ROWS_TC, NFEAT), lambda i: (i, 0)),
            pl.BlockSpec((NFEAT, NHID), lambda i: (0, 0)),
            pl.BlockSpec((NC, ROWS_TC, 16), lambda i: (0, i, 0)),
        ],
        out_specs=[
            pl.BlockSpec((ROWS_TC, NHID), lambda i: (i, 0)),
            pl.BlockSpec((ROWS_TC, 16), lambda i: (i, 0)),
        ],
        out_shape=[
            jax.ShapeDtypeStruct((N_NODES, NHID), jnp.float32),
            jax.ShapeDtypeStruct((N_NODES, 16), jnp.float32),
        ],
    )(x, W1, degc)

    p = _spmm_dbg(xws, src3, dst3, NHID)

    hs = pl.pallas_call(
        _mid_body,
        grid=grid,
        in_specs=[
            pl.BlockSpec((NC, ROWS_TC, NHID), lambda i: (0, i, 0)),
            pl.BlockSpec((ROWS_TC, 16), lambda i: (i, 0)),
        ],
        out_specs=pl.BlockSpec((ROWS_TC, NHID), lambda i: (i, 0)),
        out_shape=jax.ShapeDtypeStruct((N_NODES, NHID), jnp.float32),
    )(p, dinv16)

    q = _spmm_dbg(hs, src3, dst3, NHID)

    out = pl.pallas_call(
        _final_body,
        grid=grid,
        in_specs=[
            pl.BlockSpec((NC, ROWS_TC, NHID), lambda i: (0, i, 0)),
            pl.BlockSpec((ROWS_TC, 16), lambda i: (i, 0)),
            pl.BlockSpec((NHID, NCLASS), lambda i: (0, 0)),
            pl.BlockSpec((1, NCLASS), lambda i: (0, 0)),
        ],
        out_specs=pl.BlockSpec((ROWS_TC, NCLASS), lambda i: (i, 0)),
        out_shape=jax.ShapeDtypeStruct((N_NODES, NCLASS), jnp.float32),
    )(q, dinv16, W2, b2r)
    return out
